# baseline (speedup 1.0000x reference)
import functools

import jax
import jax.numpy as jnp
import numpy as np
from jax import lax
from jax.experimental import pallas as pl
from jax.experimental.pallas import tpu as pltpu
from jax.experimental.pallas import tpu_sc as plsc

T, D, K, H, W = 128, 768, 16, 512, 512
HW = H * W
NC, NS, L = 2, 16, 16
NW = NC * NS
TSC = 96
NT = T - TSC
FPW = TSC // NW
CH = 32768
NCHUNK = HW // CH
HBINS = (K + 1) * L


CROWS = CH // W
NPAIR = K + 1
HBINS2 = NPAIR * NPAIR * L


def _sc_hist(mask_hbm, out_hbm, buf0, buf1, hist, sem0, sem1):
    wid = lax.axis_index("s") * NC + lax.axis_index("c")
    lane = lax.iota(jnp.int32, L)
    ones = jnp.ones((L,), jnp.float32)
    zeros = jnp.zeros((L,), jnp.float32)
    bufs = (buf0, buf1)
    sems = (sem0, sem1)
    for f in range(FPW):
        frame = wid * FPW + f

        @plsc.parallel_loop(0, HBINS2 // L, unroll=4)
        def _(z):
            hist[pl.ds(z * L, L)] = zeros

        desc = [None, None]
        desc[0] = pltpu.async_copy(
            mask_hbm.at[frame, pl.ds(0, CROWS)], buf0, sem0)
        for c in range(NCHUNK):
            if c + 1 < NCHUNK:
                desc[(c + 1) % 2] = pltpu.async_copy(
                    mask_hbm.at[frame, pl.ds((c + 1) * CROWS, CROWS)],
                    bufs[(c + 1) % 2], sems[(c + 1) % 2])
            desc[c % 2].wait()
            buf = bufs[c % 2]

            @plsc.parallel_loop(0, CROWS * 4, unroll=2)
            def _(i):
                r = i >> 2
                col = (i & 3) << 7
                for cc in range(4):
                    a = buf[r, pl.ds(col + cc * 2 * L, L)]
                    b = buf[r, pl.ds(col + cc * 2 * L + L, L)]
                    idx = ((((a << 4) + a + b) << 4) | lane)
                    plsc.addupdate_scatter(hist, [idx], ones)
        pltpu.sync_copy(hist, out_hbm.at[frame])


def _tc_hist_body(mask_ref, counts_ref):
    t = pl.program_id(0)
    m = mask_ref[...]
    sums = []
    for k in range(1, K + 1):
        sums.append(jnp.sum((m == k).astype(jnp.float32)))
    counts_ref[t] = jnp.stack(sums)


def _dense_body(ff_ref, marks_ref, w_ref, b_ref, gamma_ref, hist_ref, fold_ref,
                counts_tc_ref, out_ref):
    counts_sc = jax.lax.dot_general(
        hist_ref[...], fold_ref[...], (((1,), (0,)), ((), ())),
        preferred_element_type=jnp.float32)
    counts = jnp.concatenate([counts_sc, counts_tc_ref[...]], axis=0)
    p = jax.lax.dot_general(
        marks_ref[...], w_ref[...], (((1,), (1,)), ((), ())),
        preferred_element_type=jnp.float32)
    p = p + b_ref[...]
    sm = jax.lax.dot_general(
        counts, p, (((1,), (0,)), ((), ())),
        preferred_element_type=jnp.float32)
    wsum = jnp.sum(counts, axis=1, keepdims=True) + 1e-6
    out_ref[...] = ff_ref[...] + gamma_ref[0] * sm / wsum


_FOLD = np.zeros((HBINS2, K), np.float32)
for _a in range(NPAIR):
    for _b in range(NPAIR):
        _r0 = (_a * NPAIR + _b) * L
        if _a >= 1:
            _FOLD[_r0:_r0 + L, _a - 1] += 1.0
        if _b >= 1:
            _FOLD[_r0:_r0 + L, _b - 1] += 1.0


@jax.jit
def kernel(frame_feat, mark_embeddings, W_frame, b_frame, gamma, frame_masks):
    sc_hist = pl.kernel(
        _sc_hist,
        out_type=jax.ShapeDtypeStruct((TSC, HBINS2), jnp.float32),
        mesh=plsc.VectorSubcoreMesh(core_axis_name="c", subcore_axis_name="s"),
        scratch_types=[
            pltpu.VMEM((CROWS, W), jnp.int32),
            pltpu.VMEM((CROWS, W), jnp.int32),
            pltpu.VMEM((HBINS2,), jnp.float32),
            pltpu.SemaphoreType.DMA,
            pltpu.SemaphoreType.DMA,
        ],
        compiler_params=pltpu.CompilerParams(needs_layout_passes=False),
    )
    hist_all = sc_hist(frame_masks)

    counts_tc = pl.pallas_call(
        _tc_hist_body,
        grid=(NT,),
        in_specs=[pl.BlockSpec((1, H, W), lambda t: (t + TSC, 0, 0))],
        out_specs=pl.BlockSpec((NT, K), lambda t: (0, 0)),
        out_shape=jax.ShapeDtypeStruct((NT, K), jnp.float32),
    )(frame_masks)

    out = pl.pallas_call(
        _dense_body,
        in_specs=[
            pl.BlockSpec((T, D), lambda: (0, 0)),
            pl.BlockSpec((K, D), lambda: (0, 0)),
            pl.BlockSpec((D, D), lambda: (0, 0)),
            pl.BlockSpec((1, D), lambda: (0, 0)),
            pl.BlockSpec(memory_space=pltpu.SMEM),
            pl.BlockSpec((TSC, HBINS2), lambda: (0, 0)),
            pl.BlockSpec((HBINS2, K), lambda: (0, 0)),
            pl.BlockSpec((NT, K), lambda: (0, 0)),
        ],
        out_specs=pl.BlockSpec((T, D), lambda: (0, 0)),
        out_shape=jax.ShapeDtypeStruct((T, D), jnp.float32),
    )(frame_feat, mark_embeddings, W_frame, b_frame.reshape(1, D),
      jnp.reshape(gamma, (1,)), hist_all, jnp.asarray(_FOLD), counts_tc)
    return out


# speedup vs baseline: 1.0466x; 1.0466x over previous
import functools

import jax
import jax.numpy as jnp
import numpy as np
from jax import lax
from jax.experimental import pallas as pl
from jax.experimental.pallas import tpu as pltpu
from jax.experimental.pallas import tpu_sc as plsc

T, D, K, H, W = 128, 768, 16, 512, 512
HW = H * W
NC, NS, L = 2, 16, 16
NW = NC * NS
TSC = 96
NT = T - TSC
FPW = TSC // NW
CH = 32768
NCHUNK = HW // CH
HBINS = (K + 1) * L


CROWS = CH // W
NPAIR = K + 1
HBINS2 = NPAIR * NPAIR * L


def _sc_hist(mask_hbm, out_hbm, buf0, buf1, hist, sem0, sem1):
    wid = lax.axis_index("s") * NC + lax.axis_index("c")
    lane = lax.iota(jnp.int32, L)
    ones = jnp.ones((L,), jnp.float32)
    zeros = jnp.zeros((L,), jnp.float32)
    bufs = (buf0, buf1)
    sems = (sem0, sem1)
    for f in range(FPW):
        frame = wid * FPW + f

        @plsc.parallel_loop(0, HBINS2 // L, unroll=4)
        def _(z):
            hist[pl.ds(z * L, L)] = zeros

        desc = [None, None]
        desc[0] = pltpu.async_copy(
            mask_hbm.at[frame, pl.ds(0, CROWS)], buf0, sem0)
        for c in range(NCHUNK):
            if c + 1 < NCHUNK:
                desc[(c + 1) % 2] = pltpu.async_copy(
                    mask_hbm.at[frame, pl.ds((c + 1) * CROWS, CROWS)],
                    bufs[(c + 1) % 2], sems[(c + 1) % 2])
            desc[c % 2].wait()
            buf = bufs[c % 2]

            @plsc.parallel_loop(0, CH // (2 * L), unroll=8)
            def _(i):
                r = i >> 4
                col = (i & 15) << 5
                a = buf[r, pl.ds(col, L)]
                b = buf[r, pl.ds(col + L, L)]
                idx = ((((a << 4) + a + b) << 4) | lane)
                plsc.addupdate_scatter(hist, [idx], ones)
        pltpu.sync_copy(hist, out_hbm.at[frame])


def _tc_hist_body(mask_ref, counts_ref):
    t = pl.program_id(0)
    m = mask_ref[...]
    sums = []
    for k in range(1, K + 1):
        sums.append(jnp.sum((m == k).astype(jnp.float32)))
    counts_ref[t] = jnp.stack(sums)


def _dense_body(ff_ref, marks_ref, w_ref, b_ref, gamma_ref, hist_ref, fold_ref,
                counts_tc_ref, out_ref):
    counts_sc = jax.lax.dot_general(
        hist_ref[...], fold_ref[...], (((1,), (0,)), ((), ())),
        preferred_element_type=jnp.float32)
    counts = jnp.concatenate([counts_sc, counts_tc_ref[...]], axis=0)
    p = jax.lax.dot_general(
        marks_ref[...], w_ref[...], (((1,), (1,)), ((), ())),
        preferred_element_type=jnp.float32)
    p = p + b_ref[...]
    sm = jax.lax.dot_general(
        counts, p, (((1,), (0,)), ((), ())),
        preferred_element_type=jnp.float32)
    wsum = jnp.sum(counts, axis=1, keepdims=True) + 1e-6
    out_ref[...] = ff_ref[...] + gamma_ref[0] * sm / wsum


_FOLD = np.zeros((HBINS2, K), np.float32)
for _a in range(NPAIR):
    for _b in range(NPAIR):
        _r0 = (_a * NPAIR + _b) * L
        if _a >= 1:
            _FOLD[_r0:_r0 + L, _a - 1] += 1.0
        if _b >= 1:
            _FOLD[_r0:_r0 + L, _b - 1] += 1.0


@jax.jit
def kernel(frame_feat, mark_embeddings, W_frame, b_frame, gamma, frame_masks):
    sc_hist = pl.kernel(
        _sc_hist,
        out_type=jax.ShapeDtypeStruct((TSC, HBINS2), jnp.float32),
        mesh=plsc.VectorSubcoreMesh(core_axis_name="c", subcore_axis_name="s"),
        scratch_types=[
            pltpu.VMEM((CROWS, W), jnp.int32),
            pltpu.VMEM((CROWS, W), jnp.int32),
            pltpu.VMEM((HBINS2,), jnp.float32),
            pltpu.SemaphoreType.DMA,
            pltpu.SemaphoreType.DMA,
        ],
        compiler_params=pltpu.CompilerParams(needs_layout_passes=False),
    )
    hist_all = sc_hist(frame_masks)

    counts_tc = pl.pallas_call(
        _tc_hist_body,
        grid=(NT,),
        in_specs=[pl.BlockSpec((1, H, W), lambda t: (t + TSC, 0, 0))],
        out_specs=pl.BlockSpec((NT, K), lambda t: (0, 0)),
        out_shape=jax.ShapeDtypeStruct((NT, K), jnp.float32),
    )(frame_masks)

    out = pl.pallas_call(
        _dense_body,
        in_specs=[
            pl.BlockSpec((T, D), lambda: (0, 0)),
            pl.BlockSpec((K, D), lambda: (0, 0)),
            pl.BlockSpec((D, D), lambda: (0, 0)),
            pl.BlockSpec((1, D), lambda: (0, 0)),
            pl.BlockSpec(memory_space=pltpu.SMEM),
            pl.BlockSpec((TSC, HBINS2), lambda: (0, 0)),
            pl.BlockSpec((HBINS2, K), lambda: (0, 0)),
            pl.BlockSpec((NT, K), lambda: (0, 0)),
        ],
        out_specs=pl.BlockSpec((T, D), lambda: (0, 0)),
        out_shape=jax.ShapeDtypeStruct((T, D), jnp.float32),
    )(frame_feat, mark_embeddings, W_frame, b_frame.reshape(1, D),
      jnp.reshape(gamma, (1,)), hist_all, jnp.asarray(_FOLD), counts_tc)
    return out


# speedup vs baseline: 1.0799x; 1.0318x over previous
import functools

import jax
import jax.numpy as jnp
import numpy as np
from jax import lax
from jax.experimental import pallas as pl
from jax.experimental.pallas import tpu as pltpu
from jax.experimental.pallas import tpu_sc as plsc

T, D, K, H, W = 128, 768, 16, 512, 512
HW = H * W
NC, NS, L = 2, 16, 16
NW = NC * NS
TSC = 96
NT = T - TSC
FPW = TSC // NW
CH = 32768
NCHUNK = HW // CH
HBINS = (K + 1) * L


CROWS = CH // W
NPAIR = K + 1
HBINS2 = NPAIR * NPAIR * L


def _sc_hist(mask_hbm, out_hbm, buf0, buf1, hist, sem0, sem1):
    wid = lax.axis_index("s") * NC + lax.axis_index("c")
    lane = lax.iota(jnp.int32, L)
    ones = jnp.ones((L,), jnp.float32)
    zeros = jnp.zeros((L,), jnp.float32)
    bufs = (buf0, buf1)
    sems = (sem0, sem1)
    NG = FPW * NCHUNK
    desc = [None, None]
    desc[0] = pltpu.async_copy(
        mask_hbm.at[wid * FPW, pl.ds(0, CROWS)], buf0, sem0)
    for g in range(NG):
        frame = wid * FPW + g // NCHUNK
        if g + 1 < NG:
            nf = wid * FPW + (g + 1) // NCHUNK
            nc = (g + 1) % NCHUNK
            desc[(g + 1) % 2] = pltpu.async_copy(
                mask_hbm.at[nf, pl.ds(nc * CROWS, CROWS)],
                bufs[(g + 1) % 2], sems[(g + 1) % 2])
        if g % NCHUNK == 0:

            @plsc.parallel_loop(0, HBINS2 // L, unroll=4)
            def _(z):
                hist[pl.ds(z * L, L)] = zeros

        desc[g % 2].wait()
        buf = bufs[g % 2]

        @plsc.parallel_loop(0, CH // (2 * L), unroll=8)
        def _(i):
            r = i >> 4
            col = (i & 15) << 5
            a = buf[r, pl.ds(col, L)]
            b = buf[r, pl.ds(col + L, L)]
            idx = ((((a << 4) + a + b) << 4) | lane)
            plsc.addupdate_scatter(hist, [idx], ones)

        if g % NCHUNK == NCHUNK - 1:
            pltpu.sync_copy(hist, out_hbm.at[frame])


def _tc_hist_body(mask_ref, counts_ref):
    t = pl.program_id(0)
    m = mask_ref[...]
    sums = []
    for k in range(1, K + 1):
        sums.append(jnp.sum((m == k).astype(jnp.float32)))
    counts_ref[t] = jnp.stack(sums)


def _dense_body(ff_ref, marks_ref, w_ref, b_ref, gamma_ref, hist_ref, fold_ref,
                counts_tc_ref, out_ref):
    counts_sc = jax.lax.dot_general(
        hist_ref[...], fold_ref[...], (((1,), (0,)), ((), ())),
        preferred_element_type=jnp.float32)
    counts = jnp.concatenate([counts_sc, counts_tc_ref[...]], axis=0)
    p = jax.lax.dot_general(
        marks_ref[...], w_ref[...], (((1,), (1,)), ((), ())),
        preferred_element_type=jnp.float32)
    p = p + b_ref[...]
    sm = jax.lax.dot_general(
        counts, p, (((1,), (0,)), ((), ())),
        preferred_element_type=jnp.float32)
    wsum = jnp.sum(counts, axis=1, keepdims=True) + 1e-6
    out_ref[...] = ff_ref[...] + gamma_ref[0] * sm / wsum


_FOLD = np.zeros((HBINS2, K), np.float32)
for _a in range(NPAIR):
    for _b in range(NPAIR):
        _r0 = (_a * NPAIR + _b) * L
        if _a >= 1:
            _FOLD[_r0:_r0 + L, _a - 1] += 1.0
        if _b >= 1:
            _FOLD[_r0:_r0 + L, _b - 1] += 1.0


@jax.jit
def kernel(frame_feat, mark_embeddings, W_frame, b_frame, gamma, frame_masks):
    sc_hist = pl.kernel(
        _sc_hist,
        out_type=jax.ShapeDtypeStruct((TSC, HBINS2), jnp.float32),
        mesh=plsc.VectorSubcoreMesh(core_axis_name="c", subcore_axis_name="s"),
        scratch_types=[
            pltpu.VMEM((CROWS, W), jnp.int32),
            pltpu.VMEM((CROWS, W), jnp.int32),
            pltpu.VMEM((HBINS2,), jnp.float32),
            pltpu.SemaphoreType.DMA,
            pltpu.SemaphoreType.DMA,
        ],
        compiler_params=pltpu.CompilerParams(needs_layout_passes=False),
    )
    hist_all = sc_hist(frame_masks)

    counts_tc = pl.pallas_call(
        _tc_hist_body,
        grid=(NT,),
        in_specs=[pl.BlockSpec((1, H, W), lambda t: (t + TSC, 0, 0))],
        out_specs=pl.BlockSpec((NT, K), lambda t: (0, 0)),
        out_shape=jax.ShapeDtypeStruct((NT, K), jnp.float32),
    )(frame_masks)

    out = pl.pallas_call(
        _dense_body,
        in_specs=[
            pl.BlockSpec((T, D), lambda: (0, 0)),
            pl.BlockSpec((K, D), lambda: (0, 0)),
            pl.BlockSpec((D, D), lambda: (0, 0)),
            pl.BlockSpec((1, D), lambda: (0, 0)),
            pl.BlockSpec(memory_space=pltpu.SMEM),
            pl.BlockSpec((TSC, HBINS2), lambda: (0, 0)),
            pl.BlockSpec((HBINS2, K), lambda: (0, 0)),
            pl.BlockSpec((NT, K), lambda: (0, 0)),
        ],
        out_specs=pl.BlockSpec((T, D), lambda: (0, 0)),
        out_shape=jax.ShapeDtypeStruct((T, D), jnp.float32),
    )(frame_feat, mark_embeddings, W_frame, b_frame.reshape(1, D),
      jnp.reshape(gamma, (1,)), hist_all, jnp.asarray(_FOLD), counts_tc)
    return out
